# baseline (speedup 1.0000x reference)
import jax
import jax.numpy as jnp
from jax import lax
from jax.experimental import pallas as pl
from jax.experimental.pallas import tpu as pltpu

B, C, H, W = 16, 3, 96, 320
HW = H * W
CHW = C * H * W
CH = C * H
K = 50
KPAD = 64
DET_THR = 0.3
NEG = -1e30
NLEV = 3
NRG = CH // 8
GCH = 25
DEPTH = 10
HP = jax.lax.Precision.HIGHEST


def _body(hm_ref, reg_ref, out_ref, vv_ref, vi_ref, d_ref, sc_ref, ind_ref,
          ring_ref, sem):
    pid = pl.program_id(0)

    def fetch(j, slot):
        return pltpu.make_async_copy(
            reg_ref.at[j, pl.ds(0, GCH)], ring_ref.at[slot], sem.at[slot])

    @pl.when(pid == 0)
    def prefetch():
        for j in range(DEPTH):
            fetch(j, j).start()

    @pl.when(pid < B)
    def phase_a():
        b = pid
        x = jax.nn.sigmoid(hm_ref[0])
        neg_row = jnp.full((C, 1, W), NEG, jnp.float32)
        up = jnp.concatenate([x[:, 1:, :], neg_row], axis=1)
        dn = jnp.concatenate([neg_row, x[:, :-1, :]], axis=1)
        m = jnp.maximum(jnp.maximum(up, dn), x)
        neg_col = jnp.full((C, H, 1), NEG, jnp.float32)
        lf = jnp.concatenate([m[:, :, 1:], neg_col], axis=2)
        rt = jnp.concatenate([neg_col, m[:, :, :-1]], axis=2)
        m = jnp.maximum(jnp.maximum(lf, rt), m)
        masked = jnp.where(m == x, x, -1.0).reshape(CH, W)
        d_ref[b] = masked
        mpad = jnp.concatenate(
            [masked, jnp.full((CH, 384 - W), -3.0, jnp.float32)], axis=1)

        base8 = (lax.broadcasted_iota(jnp.int32, (8, 128), 0) * W
                 + lax.broadcasted_iota(jnp.int32, (8, 128), 1))
        lv = [jnp.full((8, 128), -3.0, jnp.float32) for _ in range(NLEV)]
        li = [jnp.zeros((8, 128), jnp.int32) for _ in range(NLEV)]
        for r in range(NRG):
            for part in range(3):
                xv = mpad[8 * r:8 * r + 8, 128 * part:128 * part + 128]
                xi = base8 + (r * 8 * W + part * 128)
                for l in range(NLEV):
                    sw = xv > lv[l]
                    if l < NLEV - 1:
                        lv[l], xv = (jnp.where(sw, xv, lv[l]),
                                     jnp.where(sw, lv[l], xv))
                        li[l], xi = (jnp.where(sw, xi, li[l]),
                                     jnp.where(sw, li[l], xi))
                    else:
                        lv[l] = jnp.where(sw, xv, lv[l])
                        li[l] = jnp.where(sw, xi, li[l])
        vv_ref[b] = jnp.concatenate(lv, axis=0)
        vi_ref[b] = jnp.concatenate(li, axis=0)

    @pl.when(pid == B)
    def phase_b():
        vv0 = vv_ref[...]
        vi0 = vi_ref[...]
        iota_k64 = lax.broadcasted_iota(jnp.int32, (B, KPAD), 1)
        sc_init = jnp.full((B, KPAD), -3.0, jnp.float32)
        ind_init = jnp.zeros((B, KPAD), jnp.int32)

        def ext(k, c):
            l1v, l1i, l2v, l2i, l3v, kcnt, sc_a, ind_a, flag = c
            gmax = jnp.max(l1v, axis=(1, 2), keepdims=True)
            hit = l1v == gmax
            mini = jnp.min(jnp.where(hit, l1i, CHW), axis=(1, 2),
                           keepdims=True)
            kill = hit & (l1i == mini)
            flag = flag | jnp.any(kill & (kcnt == NLEV - 1)).astype(jnp.int32)
            kcnt = kcnt + kill.astype(jnp.int32)
            l1v = jnp.where(kill, l2v, l1v)
            l1i = jnp.where(kill, l2i, l1i)
            l2v = jnp.where(kill, l3v, l2v)
            l2i = jnp.where(kill, vi0[:, 16:24], l2i)
            l3v = jnp.where(kill, -3.0, l3v)
            sc_a = jnp.where(iota_k64 == k, gmax[:, :, 0], sc_a)
            ind_a = jnp.where(iota_k64 == k, mini[:, :, 0], ind_a)
            return l1v, l1i, l2v, l2i, l3v, kcnt, sc_a, ind_a, flag

        init = (vv0[:, 0:8], vi0[:, 0:8], vv0[:, 8:16], vi0[:, 8:16],
                vv0[:, 16:24], jnp.zeros((B, 8, 128), jnp.int32),
                sc_init, ind_init, jnp.int32(0))
        out = lax.fori_loop(0, K, ext, init)
        sc_fast, ind_fast, flag = out[6], out[7], out[8]

        def emit(sc_all, ind_all):
            sc_ref[...] = sc_all[:, None, :]
            ind_ref[...] = ind_all[:, None, :]

        emit(sc_fast, ind_fast)

        @pl.when(flag != 0)
        def rare_exact_path():
            flatio = (lax.broadcasted_iota(jnp.int32, (CH, W), 0) * W
                      + lax.broadcasted_iota(jnp.int32, (CH, W), 1))
            io64 = lax.broadcasted_iota(jnp.int32, (1, KPAD), 1)
            bio = lax.broadcasted_iota(jnp.int32, (B, 1), 0)

            def per_b(b, acc):
                sc_a, ind_a = acc
                dd = d_ref[b]

                def ext2(k, c2):
                    d_, scv, indv = c2
                    g = jnp.max(d_)
                    f = jnp.min(jnp.where(d_ == g, flatio, CHW))
                    d_ = jnp.where(flatio == f, -3.0, d_)
                    scv = jnp.where(io64 == k, g, scv)
                    indv = jnp.where(io64 == k, f, indv)
                    return d_, scv, indv

                _, scv, indv = lax.fori_loop(
                    0, K, ext2,
                    (dd, jnp.full((1, KPAD), -3.0, jnp.float32),
                     jnp.zeros((1, KPAD), jnp.int32)))
                rowm = bio == b
                sc_a = jnp.where(rowm, scv, sc_a)
                ind_a = jnp.where(rowm, indv, ind_a)
                return sc_a, ind_a

            sc_all, ind_all = lax.fori_loop(0, B, per_b,
                                            (sc_init, ind_init))
            emit(sc_all, ind_all)

    @pl.when(pid > B)
    def phase_c():
        b = pid - (B + 1)
        slot = lax.rem(b, DEPTH)
        fetch(b, slot).wait()
        sc = sc_ref[b]
        ind = ind_ref[b]
        spatial = ind % HW
        ys = spatial // W
        xs = spatial % W
        cls = ind // HW
        meta = jnp.concatenate(
            [xs.astype(jnp.float32), ys.astype(jnp.float32),
             cls.astype(jnp.float32), sc], axis=0)
        eye = (lax.broadcasted_iota(jnp.int32, (KPAD, KPAD), 0)
               == lax.broadcasted_iota(jnp.int32, (KPAD, KPAD), 1)
               ).astype(jnp.float32)
        dnt = (((1,), (1,)), ((), ()))
        metat = jax.lax.dot_general(eye, meta, dnt, precision=HP,
                                    preferred_element_type=jnp.float32)
        u2 = (lax.broadcasted_iota(jnp.int32, (H, KPAD), 0)
              == ys).astype(jnp.float32)
        xsti = metat[:, 0:1].astype(jnp.int32)
        v2t = (lax.broadcasted_iota(jnp.int32, (KPAD, W), 1)
               == xsti).astype(jnp.float32)
        r = ring_ref[slot]
        dny = (((0,), (0,)), ((), ()))
        pois_cols = []
        for c in range(GCH):
            s_c = jax.lax.dot_general(u2, r[c], dny, precision=HP,
                                      preferred_element_type=jnp.float32)
            pois_cols.append(jnp.sum(s_c * v2t, axis=1, keepdims=True))
        poist = jnp.concatenate(pois_cols, axis=1)
        valid = (metat[:, 3:4] >= DET_THR).astype(jnp.float32)
        cols = jnp.concatenate(
            [metat[:, 0:2], jnp.maximum(poist[:, 0:4], 0.0), poist[:, 4:25],
             metat[:, 3:4], metat[:, 2:3]], axis=1)
        out_ref[0] = (cols * valid)[:K]

        nxt = b - 1 + DEPTH
        @pl.when((b >= 1) & (nxt < B))
        def refill():
            fetch(nxt, lax.rem(nxt, DEPTH)).start()


def kernel(pred_heatmap, pred_regression):
    res = pl.pallas_call(
        _body,
        grid=(2 * B + 1,),
        in_specs=[
            pl.BlockSpec((1, C, H, W),
                         lambda i: (jnp.minimum(i, B - 1), 0, 0, 0)),
            pl.BlockSpec(memory_space=pltpu.MemorySpace.HBM),
        ],
        out_specs=pl.BlockSpec((1, K, 29),
                               lambda i: (jnp.clip(i - B - 1, 0, B - 1), 0, 0)),
        out_shape=jax.ShapeDtypeStruct((B, K, 29), jnp.float32),
        scratch_shapes=[
            pltpu.VMEM((B, NLEV * 8, 128), jnp.float32),
            pltpu.VMEM((B, NLEV * 8, 128), jnp.int32),
            pltpu.VMEM((B, CH, W), jnp.float32),
            pltpu.VMEM((B, 1, KPAD), jnp.float32),
            pltpu.VMEM((B, 1, KPAD), jnp.int32),
            pltpu.VMEM((DEPTH, GCH, H, W), jnp.float32),
            pltpu.SemaphoreType.DMA((DEPTH,)),
        ],
    )(pred_heatmap, pred_regression)
    return res.reshape(B * K, 29)


# speedup vs baseline: 1.1004x; 1.1004x over previous
import jax
import jax.numpy as jnp
from jax import lax
from jax.experimental import pallas as pl
from jax.experimental.pallas import tpu as pltpu

B, C, H, W = 16, 3, 96, 320
HW = H * W
CHW = C * H * W
CH = C * H
K = 50
KPAD = 64
DET_THR = 0.3
NEG = -1e30
NLEV = 3
NRG = CH // 8


def _topk_body(hm_ref, scores_ref, inds_ref, vv_ref, vi_ref, d_ref):
    pid = pl.program_id(0)

    @pl.when(pid < B)
    def phase_a():
        b = pid
        x = jax.nn.sigmoid(hm_ref[0])
        neg_row = jnp.full((C, 1, W), NEG, jnp.float32)
        up = jnp.concatenate([x[:, 1:, :], neg_row], axis=1)
        dn = jnp.concatenate([neg_row, x[:, :-1, :]], axis=1)
        m = jnp.maximum(jnp.maximum(up, dn), x)
        neg_col = jnp.full((C, H, 1), NEG, jnp.float32)
        lf = jnp.concatenate([m[:, :, 1:], neg_col], axis=2)
        rt = jnp.concatenate([neg_col, m[:, :, :-1]], axis=2)
        m = jnp.maximum(jnp.maximum(lf, rt), m)
        masked = jnp.where(m == x, x, -1.0).reshape(CH, W)
        d_ref[b] = masked
        mpad = jnp.concatenate(
            [masked, jnp.full((CH, 384 - W), -3.0, jnp.float32)], axis=1)

        base8 = (lax.broadcasted_iota(jnp.int32, (8, 128), 0) * W
                 + lax.broadcasted_iota(jnp.int32, (8, 128), 1))
        lv = [jnp.full((8, 128), -3.0, jnp.float32) for _ in range(NLEV)]
        li = [jnp.zeros((8, 128), jnp.int32) for _ in range(NLEV)]
        for r in range(NRG):
            for part in range(3):
                xv = mpad[8 * r:8 * r + 8, 128 * part:128 * part + 128]
                xi = base8 + (r * 8 * W + part * 128)
                for l in range(NLEV):
                    sw = xv > lv[l]
                    if l < NLEV - 1:
                        lv[l], xv = (jnp.where(sw, xv, lv[l]),
                                     jnp.where(sw, lv[l], xv))
                        li[l], xi = (jnp.where(sw, xi, li[l]),
                                     jnp.where(sw, li[l], xi))
                    else:
                        lv[l] = jnp.where(sw, xv, lv[l])
                        li[l] = jnp.where(sw, xi, li[l])
        vv_ref[b] = jnp.concatenate(lv, axis=0)
        vi_ref[b] = jnp.concatenate(li, axis=0)

    @pl.when(pid == B)
    def phase_b():
        vv0 = vv_ref[...]
        vi0 = vi_ref[...]
        iota_k64 = lax.broadcasted_iota(jnp.int32, (B, KPAD), 1)
        sc_init = jnp.full((B, KPAD), -3.0, jnp.float32)
        ind_init = jnp.zeros((B, KPAD), jnp.int32)

        def ext(k, c):
            l1v, l1i, l2v, l2i, l3v, kcnt, sc_a, ind_a, flag = c
            gmax = jnp.max(l1v, axis=(1, 2), keepdims=True)
            hit = l1v == gmax
            mini = jnp.min(jnp.where(hit, l1i, CHW), axis=(1, 2),
                           keepdims=True)
            kill = hit & (l1i == mini)
            flag = flag | jnp.any(kill & (kcnt == NLEV - 1)).astype(jnp.int32)
            kcnt = kcnt + kill.astype(jnp.int32)
            l1v = jnp.where(kill, l2v, l1v)
            l1i = jnp.where(kill, l2i, l1i)
            l2v = jnp.where(kill, l3v, l2v)
            l2i = jnp.where(kill, vi0[:, 16:24], l2i)
            l3v = jnp.where(kill, -3.0, l3v)
            sc_a = jnp.where(iota_k64 == k, gmax[:, :, 0], sc_a)
            ind_a = jnp.where(iota_k64 == k, mini[:, :, 0], ind_a)
            return l1v, l1i, l2v, l2i, l3v, kcnt, sc_a, ind_a, flag

        init = (vv0[:, 0:8], vi0[:, 0:8], vv0[:, 8:16], vi0[:, 8:16],
                vv0[:, 16:24], jnp.zeros((B, 8, 128), jnp.int32),
                sc_init, ind_init, jnp.int32(0))
        out = lax.fori_loop(0, K, ext, init)
        sc_fast, ind_fast, flag = out[6], out[7], out[8]

        def emit(sc_all, ind_all):
            scores_ref[...] = sc_all[:, None, :]
            inds_ref[...] = ind_all[:, None, :]

        emit(sc_fast, ind_fast)

        @pl.when(flag != 0)
        def rare_exact_path():
            flatio = (lax.broadcasted_iota(jnp.int32, (CH, W), 0) * W
                      + lax.broadcasted_iota(jnp.int32, (CH, W), 1))
            io64 = lax.broadcasted_iota(jnp.int32, (1, KPAD), 1)
            bio = lax.broadcasted_iota(jnp.int32, (B, 1), 0)

            def per_b(b, acc):
                sc_a, ind_a = acc
                dd = d_ref[b]

                def ext2(k, c2):
                    d_, scv, indv = c2
                    g = jnp.max(d_)
                    f = jnp.min(jnp.where(d_ == g, flatio, CHW))
                    d_ = jnp.where(flatio == f, -3.0, d_)
                    scv = jnp.where(io64 == k, g, scv)
                    indv = jnp.where(io64 == k, f, indv)
                    return d_, scv, indv

                _, scv, indv = lax.fori_loop(
                    0, K, ext2,
                    (dd, jnp.full((1, KPAD), -3.0, jnp.float32),
                     jnp.zeros((1, KPAD), jnp.int32)))
                rowm = bio == b
                sc_a = jnp.where(rowm, scv, sc_a)
                ind_a = jnp.where(rowm, indv, ind_a)
                return sc_a, ind_a

            sc_all, ind_all = lax.fori_loop(0, B, per_b,
                                            (sc_init, ind_init))
            emit(sc_all, ind_all)


def _topk_call(hm):
    out_shapes = (
        jax.ShapeDtypeStruct((B, 1, KPAD), jnp.float32),
        jax.ShapeDtypeStruct((B, 1, KPAD), jnp.int32),
    )
    return pl.pallas_call(
        _topk_body,
        grid=(B + 1,),
        in_specs=[pl.BlockSpec((1, C, H, W),
                               lambda i: (jnp.minimum(i, B - 1), 0, 0, 0))],
        out_specs=(
            pl.BlockSpec((B, 1, KPAD), lambda i: (0, 0, 0)),
            pl.BlockSpec((B, 1, KPAD), lambda i: (0, 0, 0)),
        ),
        out_shape=out_shapes,
        scratch_shapes=[
            pltpu.VMEM((B, NLEV * 8, 128), jnp.float32),
            pltpu.VMEM((B, NLEV * 8, 128), jnp.int32),
            pltpu.VMEM((B, CH, W), jnp.float32),
        ],
    )(hm)


GCH = 25
HP = jax.lax.Precision.HIGHEST


def _gather_asm_body(sc_ref, ind_ref, reg_ref, out_ref):
    sc = sc_ref[0]
    ind = ind_ref[0]
    spatial = ind % HW
    ys = spatial // W
    xs = spatial % W
    cls = ind // HW
    meta = jnp.concatenate(
        [xs.astype(jnp.float32), ys.astype(jnp.float32),
         cls.astype(jnp.float32), sc], axis=0)
    eye = (lax.broadcasted_iota(jnp.int32, (KPAD, KPAD), 0)
           == lax.broadcasted_iota(jnp.int32, (KPAD, KPAD), 1)).astype(jnp.float32)
    dn = (((1,), (1,)), ((), ()))
    metat = jax.lax.dot_general(eye, meta, dn, precision=HP,
                                preferred_element_type=jnp.float32)
    u2 = (lax.broadcasted_iota(jnp.int32, (H, KPAD), 0) == ys).astype(jnp.float32)
    xsti = metat[:, 0:1].astype(jnp.int32)
    v2t = (lax.broadcasted_iota(jnp.int32, (KPAD, W), 1)
           == xsti).astype(jnp.float32)
    r = reg_ref[0]
    dny = (((0,), (0,)), ((), ()))
    pois_cols = []
    for c in range(GCH):
        s_c = jax.lax.dot_general(u2, r[c], dny, precision=HP,
                                  preferred_element_type=jnp.float32)
        pois_cols.append(jnp.sum(s_c * v2t, axis=1, keepdims=True))
    poist = jnp.concatenate(pois_cols, axis=1)
    valid = (metat[:, 3:4] >= DET_THR).astype(jnp.float32)
    cols = jnp.concatenate(
        [metat[:, 0:2], jnp.maximum(poist[:, 0:4], 0.0), poist[:, 4:25],
         metat[:, 3:4], metat[:, 2:3]], axis=1)
    out_ref[0] = (cols * valid)[:K]


def _gather_asm_call(scores, inds, reg):
    return pl.pallas_call(
        _gather_asm_body,
        grid=(B,),
        in_specs=[
            pl.BlockSpec((1, 1, KPAD), lambda b: (b, 0, 0)),
            pl.BlockSpec((1, 1, KPAD), lambda b: (b, 0, 0)),
            pl.BlockSpec((1, GCH, H, W), lambda b: (b, 0, 0, 0)),
        ],
        out_specs=pl.BlockSpec((1, K, 29), lambda b: (b, 0, 0)),
        out_shape=jax.ShapeDtypeStruct((B, K, 29), jnp.float32),
    )(scores, inds, reg)


def kernel(pred_heatmap, pred_regression):
    scores, inds = _topk_call(pred_heatmap)
    res = _gather_asm_call(scores, inds, pred_regression)
    return res.reshape(B * K, 29)
